# Initial kernel scaffold; baseline (speedup 1.0000x reference)
import jax
import jax.numpy as jnp
from jax.experimental import pallas as pl


def kernel(x, edge_index, W1, att_src1, att_dst1, b1, W2, att_src2, att_dst2, b2):
    raise NotImplementedError("write your pallas kernel here")


# speedup vs baseline: 1.0353x; 1.0353x over previous
import functools

import jax
import jax.numpy as jnp
from jax.experimental import pallas as pl

N = 10000
E = 320000
HEADS = 8


def _mm_body(x_ref, w_ref, o_ref):
    o_ref[...] = jnp.dot(x_ref[...], w_ref[...],
                         preferred_element_type=jnp.float32)


def _matmul(x, w, block=1000):
    m, k = x.shape
    n = w.shape[1]
    grid = (m + block - 1) // block
    return pl.pallas_call(
        _mm_body,
        out_shape=jax.ShapeDtypeStruct((m, n), jnp.float32),
        grid=(grid,),
        in_specs=[
            pl.BlockSpec((block, k), lambda i: (i, 0)),
            pl.BlockSpec((k, n), lambda i: (0, 0)),
        ],
        out_specs=pl.BlockSpec((block, n), lambda i: (i, 0)),
    )(x, w)


def _gat_layer(x, edge_index, W, a_src, a_dst, heads, out_ch):
    n = x.shape[0]
    h = _matmul(x, W).reshape(n, heads, out_ch)
    src = edge_index[0]
    dst = edge_index[1]
    alpha_src = (h * a_src[None, :, :]).sum(-1)
    alpha_dst = (h * a_dst[None, :, :]).sum(-1)
    e = alpha_src[src] + alpha_dst[dst]
    e = jax.nn.leaky_relu(e, negative_slope=0.2)
    ex = jnp.exp(e)
    den = jax.ops.segment_sum(ex, dst, num_segments=n)
    alpha = ex / (den[dst] + 1e-16)
    msg = h[src] * alpha[:, :, None]
    out = jax.ops.segment_sum(msg, dst, num_segments=n)
    return out, alpha


def kernel(x, edge_index, W1, att_src1, att_dst1, b1,
           W2, att_src2, att_dst2, b2):
    out1, alpha1 = _gat_layer(x, edge_index, W1, att_src1, att_dst1, 8, 8)
    h1 = jax.nn.elu(out1.reshape(N, 64) + b1)
    out2, alpha2 = _gat_layer(h1, edge_index, W2, att_src2, att_dst2, 8, 16)
    h2 = out2.mean(axis=1) + b2
    out = jax.nn.log_softmax(h2, axis=-1)
    return (out, (edge_index, alpha1), (edge_index, alpha2))


# speedup vs baseline: 84.5785x; 81.6967x over previous
import functools

import jax
import jax.numpy as jnp
from jax import lax
from jax.experimental import pallas as pl
from jax.experimental.pallas import tpu as pltpu
from jax.experimental.pallas import tpu_sc as plsc

N = 10000
NP = 10240
E = 320000
D_IN = 128
H = 8
NC = 2
NS = 16
NW = NC * NS
SUB = 128
IG = 512
IGS = IG // SUB
W_PER = 10240
EP = W_PER * NW
N_TILE = NP // NS

_mesh = plsc.VectorSubcoreMesh(core_axis_name="c", subcore_axis_name="s",
                               num_cores=NC, num_subcores=NS)


def _pass1_body(B, sa_hbm, da_hbm, src3d_hbm, dst3d_hbm, ex_hbm, den_hbm,
                srcv, dstv, rows_s, rows_d, exb, den_sp, sem):
    GRP = B // SUB
    NCHUNK = W_PER // B
    cid = lax.axis_index("c")
    sid = lax.axis_index("s")
    wid = sid * NC + cid
    base_w = wid * W_PER

    if True:
        @plsc.parallel_loop(0, N_TILE, unroll=4)
        def _z(r):
            exb[r, :] = jnp.zeros((16,), jnp.float32)
        pltpu.sync_copy(exb.at[pl.ds(0, N_TILE), :],
                        den_sp.at[pl.ds(sid * N_TILE, N_TILE), :])
        plsc.subcore_barrier()

        for c in range(NCHUNK):
            base = base_w + c * B
            for t in range(B // IG):
                g = base // IG + t
                pltpu.sync_copy(src3d_hbm.at[g],
                                srcv.at[pl.ds(t * IGS, IGS), :])
                pltpu.sync_copy(dst3d_hbm.at[g],
                                dstv.at[pl.ds(t * IGS, IGS), :])
            cps = []
            for j in range(GRP):
                cps.append(pltpu.async_copy(
                    sa_hbm.at[srcv.at[j]],
                    rows_s.at[pl.ds(j * SUB, SUB), :], sem))
                cps.append(pltpu.async_copy(
                    da_hbm.at[dstv.at[j]],
                    rows_d.at[pl.ds(j * SUB, SUB), :], sem))
            for cp in cps:
                cp.wait()

            @plsc.parallel_loop(0, B, unroll=8)
            def _edge(k):
                e16 = rows_s[k, :] + rows_d[k, :]
                e16 = jnp.maximum(e16, e16 * 0.2)
                ex16 = jnp.exp(e16)
                valid = (base + k) < E
                exb[k, :] = jnp.where(valid, ex16, 0.0)

            pltpu.sync_copy(exb.at[pl.ds(0, B), :],
                            ex_hbm.at[pl.ds(base, B), :])
            cps = []
            for j in range(GRP):
                cps.append(pltpu.async_copy(
                    exb.at[pl.ds(j * SUB, SUB), :],
                    den_sp.at[dstv.at[j]], sem, add=True))
            for cp in cps:
                cp.wait()

        plsc.subcore_barrier()
        pltpu.sync_copy(den_sp.at[pl.ds(sid * N_TILE, N_TILE), :],
                        den_hbm.at[cid, pl.ds(sid * N_TILE, N_TILE), :])


def _pass1(sa, da, src3d, dst3d):
    B = 1024
    kern = pl.kernel(
        functools.partial(_pass1_body, B),
        out_type=[jax.ShapeDtypeStruct((EP, 16), jnp.float32),
                  jax.ShapeDtypeStruct((NC, NP, 16), jnp.float32)],
        mesh=_mesh,
        compiler_params=pltpu.CompilerParams(use_tc_tiling_on_sc=False, needs_layout_passes=False),
        scratch_types=[
            pltpu.VMEM((B // SUB, SUB), jnp.int32),
            pltpu.VMEM((B // SUB, SUB), jnp.int32),
            pltpu.VMEM((B, 16), jnp.float32),
            pltpu.VMEM((B, 16), jnp.float32),
            pltpu.VMEM((B, 16), jnp.float32),
            pltpu.VMEM_SHARED((NP, 16), jnp.float32),
            pltpu.SemaphoreType.DMA,
        ],
    )
    return kern(sa, da, src3d, dst3d)


def _pass2_body(hc, B, epc, h_hbm, inv_hbm, ex_hbm, src3d_hbm, dst3d_hbm,
                alpha_hbm, acc_hbm,
                srcv, dstv, hrows, invb, exb, alb, acc_sp, sem):
    nh = hc // 16
    GRP = B // SUB
    NCHUNK = W_PER // B
    nq = B // epc
    sgrp = epc // SUB
    cid = lax.axis_index("c")
    sid = lax.axis_index("s")
    wid = sid * NC + cid
    base_w = wid * W_PER
    lane = lax.iota(jnp.int32, 16)
    lane_mask = lane < 8
    zrows = N_TILE // 5

    if True:
        @plsc.parallel_loop(0, zrows, unroll=2)
        def _z(r):
            for j in range(nh):
                hrows[r, pl.ds(j * 16, 16)] = jnp.zeros((16,), jnp.float32)
        for t in range(5):
            pltpu.sync_copy(
                hrows.at[pl.ds(0, zrows), :],
                acc_sp.at[pl.ds(sid * N_TILE + t * zrows, zrows), :])
        plsc.subcore_barrier()

        for c in range(NCHUNK):
            base = base_w + c * B
            for t in range(B // IG):
                g = base // IG + t
                pltpu.sync_copy(src3d_hbm.at[g],
                                srcv.at[pl.ds(t * IGS, IGS), :])
                pltpu.sync_copy(dst3d_hbm.at[g],
                                dstv.at[pl.ds(t * IGS, IGS), :])
            pltpu.sync_copy(ex_hbm.at[pl.ds(base, B), :], exb)
            cps = []
            for j in range(GRP):
                cps.append(pltpu.async_copy(
                    inv_hbm.at[dstv.at[j]],
                    invb.at[pl.ds(j * SUB, SUB), :], sem))
            for cp in cps:
                cp.wait()

            for q in range(nq):
                cps = []
                for j in range(sgrp):
                    cps.append(pltpu.async_copy(
                        h_hbm.at[srcv.at[q * sgrp + j]],
                        hrows.at[pl.ds(j * SUB, SUB), :], sem))
                for cp in cps:
                    cp.wait()

                @plsc.parallel_loop(0, epc, unroll=2)
                def _edge(k):
                    ke = q * epc + k
                    al16 = exb[ke, :] * invb[ke, :]
                    plsc.store_scatter(alb, [ke * 8 + lane], al16,
                                       mask=lane_mask)
                    for j in range(nh):
                        if hc == 128:
                            sc = lax.broadcast(al16[j], (16,))
                        else:
                            sc = lax.broadcast(al16[2 * j], (16,))
                            sc2 = lax.broadcast(al16[2 * j + 1], (16,))
                            sc = jnp.where(lane_mask, sc, sc2)
                        hrows[k, pl.ds(j * 16, 16)] = \
                            hrows[k, pl.ds(j * 16, 16)] * sc

                cps = []
                for j in range(sgrp):
                    cps.append(pltpu.async_copy(
                        hrows.at[pl.ds(j * SUB, SUB), :],
                        acc_sp.at[dstv.at[q * sgrp + j]], sem, add=True))
                for cp in cps:
                    cp.wait()

            pltpu.sync_copy(alb, alpha_hbm.at[pl.ds(base * 8, B * 8)])

        plsc.subcore_barrier()
        pltpu.sync_copy(acc_sp.at[pl.ds(sid * N_TILE, N_TILE), :],
                        acc_hbm.at[cid, pl.ds(sid * N_TILE, N_TILE), :])


def _pass2(hc, h, inv, ex, src3d, dst3d):
    B = 1024 if hc == 64 else 512
    epc = (4 * SUB) if hc == 64 else (1 * SUB)
    kern = pl.kernel(
        functools.partial(_pass2_body, hc, B, epc),
        out_type=[jax.ShapeDtypeStruct((EP * 8,), jnp.float32),
                  jax.ShapeDtypeStruct((NC, NP, hc), jnp.float32)],
        mesh=_mesh,
        compiler_params=pltpu.CompilerParams(use_tc_tiling_on_sc=False, needs_layout_passes=False),
        scratch_types=[
            pltpu.VMEM((B // SUB, SUB), jnp.int32),
            pltpu.VMEM((B // SUB, SUB), jnp.int32),
            pltpu.VMEM((epc, hc), jnp.float32),
            pltpu.VMEM((B, 16), jnp.float32),
            pltpu.VMEM((B, 16), jnp.float32),
            pltpu.VMEM((B * 8,), jnp.float32),
            pltpu.VMEM_SHARED((NP, hc), jnp.float32),
            pltpu.SemaphoreType.DMA,
        ],
    )
    return kern(h, inv, ex, src3d, dst3d)


def _k0_body(x_ref, w_ref, asa_ref, ada_ref, h_ref, sa_ref, da_ref):
    h = jnp.dot(x_ref[...], w_ref[...], preferred_element_type=jnp.float32)
    h_ref[...] = h
    sa_ref[...] = jnp.dot(h, asa_ref[...], preferred_element_type=jnp.float32)
    da_ref[...] = jnp.dot(h, ada_ref[...], preferred_element_type=jnp.float32)


def _k0(x, w, asa, ada):
    blk = 2000
    grid = (N + blk - 1) // blk
    d_in, d_out = w.shape
    return pl.pallas_call(
        _k0_body,
        out_shape=[jax.ShapeDtypeStruct((N, d_out), jnp.float32),
                   jax.ShapeDtypeStruct((N, 16), jnp.float32),
                   jax.ShapeDtypeStruct((N, 16), jnp.float32)],
        grid=(grid,),
        in_specs=[pl.BlockSpec((blk, d_in), lambda i: (i, 0)),
                  pl.BlockSpec((d_in, d_out), lambda i: (0, 0)),
                  pl.BlockSpec((d_out, 16), lambda i: (0, 0)),
                  pl.BlockSpec((d_out, 16), lambda i: (0, 0))],
        out_specs=[pl.BlockSpec((blk, d_out), lambda i: (i, 0)),
                   pl.BlockSpec((blk, 16), lambda i: (i, 0)),
                   pl.BlockSpec((blk, 16), lambda i: (i, 0))],
    )(x, w, asa, ada)


def _inv_body(den_ref, inv_ref):
    inv_ref[...] = 1.0 / (den_ref[0] + den_ref[1] + 1e-16)


def _invden(den):
    blk = 2048
    grid = NP // blk
    return pl.pallas_call(
        _inv_body,
        out_shape=jax.ShapeDtypeStruct((NP, 16), jnp.float32),
        grid=(grid,),
        in_specs=[pl.BlockSpec((NC, blk, 16), lambda i: (0, i, 0))],
        out_specs=pl.BlockSpec((blk, 16), lambda i: (i, 0)),
    )(den)


def _k4_body(acc_ref, b1_ref, w2_ref, asa_ref, ada_ref,
             h2_ref, sa_ref, da_ref):
    s = acc_ref[0] + acc_ref[1] + b1_ref[...]
    h1 = jnp.where(s > 0, s, jnp.exp(jnp.minimum(s, 0.0)) - 1.0)
    h2 = jnp.dot(h1, w2_ref[...], preferred_element_type=jnp.float32)
    h2_ref[...] = h2
    sa_ref[...] = jnp.dot(h2, asa_ref[...], preferred_element_type=jnp.float32)
    da_ref[...] = jnp.dot(h2, ada_ref[...], preferred_element_type=jnp.float32)


def _k4(acc, b1, w2, asa, ada):
    blk = 2000
    grid = (N + blk - 1) // blk
    return pl.pallas_call(
        _k4_body,
        out_shape=[jax.ShapeDtypeStruct((N, 128), jnp.float32),
                   jax.ShapeDtypeStruct((N, 16), jnp.float32),
                   jax.ShapeDtypeStruct((N, 16), jnp.float32)],
        grid=(grid,),
        in_specs=[pl.BlockSpec((NC, blk, 64), lambda i: (0, i, 0)),
                  pl.BlockSpec((1, 64), lambda i: (0, 0)),
                  pl.BlockSpec((64, 128), lambda i: (0, 0)),
                  pl.BlockSpec((128, 16), lambda i: (0, 0)),
                  pl.BlockSpec((128, 16), lambda i: (0, 0))],
        out_specs=[pl.BlockSpec((blk, 128), lambda i: (i, 0)),
                   pl.BlockSpec((blk, 16), lambda i: (i, 0)),
                   pl.BlockSpec((blk, 16), lambda i: (i, 0))],
    )(acc, b1, w2, asa, ada)


def _k5_body(acc_ref, b2_ref, out_ref):
    t = acc_ref[0] + acc_ref[1]
    s = t[:, 0:16]
    for hh in range(1, H):
        s = s + t[:, hh * 16:(hh + 1) * 16]
    s = s * (1.0 / H) + b2_ref[...]
    m = jnp.max(s, axis=-1, keepdims=True)
    lse = jnp.log(jnp.sum(jnp.exp(s - m), axis=-1, keepdims=True)) + m
    out_ref[...] = s - lse


def _k5(acc, b2):
    blk = 2000
    grid = (N + blk - 1) // blk
    return pl.pallas_call(
        _k5_body,
        out_shape=jax.ShapeDtypeStruct((N, 16), jnp.float32),
        grid=(grid,),
        in_specs=[pl.BlockSpec((NC, blk, 128), lambda i: (0, i, 0)),
                  pl.BlockSpec((1, 16), lambda i: (0, 0))],
        out_specs=pl.BlockSpec((blk, 16), lambda i: (i, 0)),
    )(acc, b2)


def _att_mats(att_src, att_dst, ch):
    hc = H * ch
    rows = jnp.arange(hc, dtype=jnp.int32)
    hd = rows // ch
    asa = jnp.zeros((hc, 16), jnp.float32)
    asa = asa.at[rows, hd].set(att_src.reshape(hc))
    asa = asa.at[rows, hd + 8].set(att_dst.reshape(hc))
    ada = jnp.zeros((hc, 16), jnp.float32)
    ada = ada.at[rows, hd].set(att_dst.reshape(hc))
    ada = ada.at[rows, hd + 8].set(att_src.reshape(hc))
    return asa, ada


def kernel(x, edge_index, W1, att_src1, att_dst1, b1,
           W2, att_src2, att_dst2, b2):
    src = edge_index[0]
    dst = edge_index[1]
    padlen = EP - E
    padidx = (jnp.arange(padlen, dtype=jnp.int32) * 37) % N
    src3d = jnp.concatenate([src, padidx]).reshape(EP // IG, IGS, SUB)
    dst3d = jnp.concatenate([dst, padidx]).reshape(EP // IG, IGS, SUB)

    asa1, ada1 = _att_mats(att_src1, att_dst1, 8)
    asa2, ada2 = _att_mats(att_src2, att_dst2, 16)

    h1, sa1, da1 = _k0(x, W1, asa1, ada1)
    ex1, den1 = _pass1(sa1, da1, src3d, dst3d)
    inv1 = _invden(den1)
    alpha1f, acc1 = _pass2(64, h1, inv1, ex1, src3d, dst3d)
    h2, sa2, da2 = _k4(acc1, b1.reshape(1, 64), W2, asa2, ada2)
    ex2, den2 = _pass1(sa2, da2, src3d, dst3d)
    inv2 = _invden(den2)
    alpha2f, acc2 = _pass2(128, h2, inv2, ex2, src3d, dst3d)
    out = _k5(acc2, b2.reshape(1, 16))

    alpha1 = alpha1f.reshape(EP, 8)[:E]
    alpha2 = alpha2f.reshape(EP, 8)[:E]
    return (out, (edge_index, alpha1), (edge_index, alpha2))


# speedup vs baseline: 107.9454x; 1.2763x over previous
import functools

import jax
import jax.numpy as jnp
from jax import lax
from jax.experimental import pallas as pl
from jax.experimental.pallas import tpu as pltpu
from jax.experimental.pallas import tpu_sc as plsc

N = 10000
NP = 10240
E = 320000
D_IN = 128
H = 8
NC = 2
NS = 16
NW = NC * NS
SUB = 128
IG = 512
IGS = IG // SUB
W_PER = 10240
EP = W_PER * NW
N_TILE = NP // NS

_mesh = plsc.VectorSubcoreMesh(core_axis_name="c", subcore_axis_name="s",
                               num_cores=NC, num_subcores=NS)
_params = pltpu.CompilerParams(use_tc_tiling_on_sc=False,
                               needs_layout_passes=False)


def _pass1_body(sa_hbm, da_hbm, src3d_hbm, dst3d_hbm, ex_hbm, den_hbm,
                srcv, dv0, dv1, rs0, rs1, rd0, rd1, ex0, ex1,
                den_sp, gsem, ssem):
    B = 1024
    GRP = B // SUB
    NCHUNK = W_PER // B
    cid = lax.axis_index("c")
    sid = lax.axis_index("s")
    wid = sid * NC + cid
    base_w = wid * W_PER
    dv = [dv0, dv1]
    rs = [rs0, rs1]
    rd = [rd0, rd1]
    exs = [ex0, ex1]

    @plsc.parallel_loop(0, N_TILE, unroll=4)
    def _z(r):
        ex0[r, :] = jnp.zeros((16,), jnp.float32)
    pltpu.sync_copy(ex0.at[pl.ds(0, N_TILE), :],
                    den_sp.at[pl.ds(sid * N_TILE, N_TILE), :])
    plsc.subcore_barrier()

    def stage_idx(c, p):
        base = base_w + c * B
        for t in range(B // IG):
            g = base // IG + t
            pltpu.sync_copy(src3d_hbm.at[g], srcv.at[pl.ds(t * IGS, IGS), :])
            pltpu.sync_copy(dst3d_hbm.at[g], dv[p].at[pl.ds(t * IGS, IGS), :])

    def fire_gathers(p):
        cps = []
        for j in range(GRP):
            cps.append(pltpu.async_copy(
                sa_hbm.at[srcv.at[j]],
                rs[p].at[pl.ds(j * SUB, SUB), :], gsem))
            cps.append(pltpu.async_copy(
                da_hbm.at[dv[p].at[j]],
                rd[p].at[pl.ds(j * SUB, SUB), :], gsem))
        return cps

    stage_idx(0, 0)
    pend_g = fire_gathers(0)
    prev_sc = [None, None]
    for c in range(NCHUNK):
        p = c & 1
        base = base_w + c * B
        for cp in pend_g:
            cp.wait()
        if c + 1 < NCHUNK:
            if prev_sc[1 - p] is not None:
                for cp in prev_sc[1 - p]:
                    cp.wait()
                prev_sc[1 - p] = None
            stage_idx(c + 1, 1 - p)
            pend_g = fire_gathers(1 - p)
        if prev_sc[p] is not None:
            for cp in prev_sc[p]:
                cp.wait()
            prev_sc[p] = None
        rsp, rdp, exp_ = rs[p], rd[p], exs[p]

        @plsc.parallel_loop(0, B, unroll=8)
        def _edge(k):
            e16 = rsp[k, :] + rdp[k, :]
            e16 = jnp.maximum(e16, e16 * 0.2)
            ex16 = jnp.exp(e16)
            valid = (base + k) < E
            exp_[k, :] = jnp.where(valid, ex16, 0.0)

        pltpu.sync_copy(exp_.at[pl.ds(0, B), :],
                        ex_hbm.at[pl.ds(base, B), :])
        cps = []
        for j in range(GRP):
            cps.append(pltpu.async_copy(
                exp_.at[pl.ds(j * SUB, SUB), :],
                den_sp.at[dv[p].at[j]], ssem, add=True))
        prev_sc[p] = cps

    for q in (0, 1):
        if prev_sc[q] is not None:
            for cp in prev_sc[q]:
                cp.wait()
    plsc.subcore_barrier()
    pltpu.sync_copy(den_sp.at[pl.ds(sid * N_TILE, N_TILE), :],
                    den_hbm.at[cid, pl.ds(sid * N_TILE, N_TILE), :])


def _pass1(sa, da, src3d, dst3d):
    B = 1024
    kern = pl.kernel(
        _pass1_body,
        out_type=[jax.ShapeDtypeStruct((EP, 16), jnp.float32),
                  jax.ShapeDtypeStruct((NC, NP, 16), jnp.float32)],
        mesh=_mesh,
        compiler_params=_params,
        scratch_types=[
            pltpu.VMEM((B // SUB, SUB), jnp.int32),
            pltpu.VMEM((B // SUB, SUB), jnp.int32),
            pltpu.VMEM((B // SUB, SUB), jnp.int32),
            pltpu.VMEM((B, 16), jnp.float32),
            pltpu.VMEM((B, 16), jnp.float32),
            pltpu.VMEM((B, 16), jnp.float32),
            pltpu.VMEM((B, 16), jnp.float32),
            pltpu.VMEM((B, 16), jnp.float32),
            pltpu.VMEM((B, 16), jnp.float32),
            pltpu.VMEM_SHARED((NP, 16), jnp.float32),
            pltpu.SemaphoreType.DMA,
            pltpu.SemaphoreType.DMA,
        ],
    )
    return kern(sa, da, src3d, dst3d)


def _pass2_body(hc, h_hbm, inv_hbm, ex_hbm, src3d_hbm, dst3d_hbm,
                alpha_hbm, acc_hbm,
                srcv, dv0, dv1, h0, h1, h2, m0, m1, invb, exb, alb,
                acc_sp, gsem, ssem, isem):
    B = 1024
    GRP = B // SUB
    NCHUNK = W_PER // B
    nh = hc // 16
    epc = 128 if hc == 64 else 256
    nq = B // epc
    sgrp = epc // SUB
    acw = 64 if hc == 64 else 16
    cid = lax.axis_index("c")
    sid = lax.axis_index("s")
    wid = sid * NC + cid
    base_w = wid * W_PER
    lane = lax.iota(jnp.int32, 16)
    lane_mask = lane < 8
    dv = [dv0, dv1]
    hbufs = [h0, h1, h2] if hc == 64 else [h0, h1]
    mbufs = [m0, m1]
    nhb = len(hbufs)
    zrows = N_TILE // 5

    @plsc.parallel_loop(0, zrows, unroll=2)
    def _z(r):
        for j in range(acw // 16):
            h0[r, pl.ds(j * 16, 16)] = jnp.zeros((16,), jnp.float32)
    for t in range(5):
        pltpu.sync_copy(
            h0.at[pl.ds(0, zrows), pl.ds(0, acw)],
            acc_sp.at[pl.ds(sid * N_TILE + t * zrows, zrows), :])
    plsc.subcore_barrier()

    prev_sc = [None] * nhb

    def wait_sc(i):
        if prev_sc[i] is not None:
            for cp in prev_sc[i]:
                cp.wait()
            prev_sc[i] = None

    def fire_gather(q, hb):
        cps = []
        for j in range(sgrp):
            cps.append(pltpu.async_copy(
                h_hbm.at[srcv.at[q * sgrp + j]],
                hbufs[hb].at[pl.ds(j * SUB, SUB), :], gsem))
        return cps

    def stage_chunk(c):
        base = base_w + c * B
        p = c & 1
        for t in range(B // IG):
            g = base // IG + t
            pltpu.sync_copy(src3d_hbm.at[g], srcv.at[pl.ds(t * IGS, IGS), :])
            pltpu.sync_copy(dst3d_hbm.at[g], dv[p].at[pl.ds(t * IGS, IGS), :])
        pltpu.sync_copy(ex_hbm.at[pl.ds(base, B), :], exb)
        cps = []
        for j in range(GRP):
            cps.append(pltpu.async_copy(
                inv_hbm.at[dv[p].at[j]],
                invb.at[pl.ds(j * SUB, SUB), :], isem))
        for cp in cps:
            cp.wait()

    hb = 0
    for c in range(NCHUNK):
        base = base_w + c * B
        stage_chunk(c)
        pend = None
        for q in range(nq):
            cur = hb
            if pend is None:
                wait_sc(cur)
                pend = fire_gather(q, cur)
            nxt = (cur + 1) % nhb
            if q + 1 < nq:
                wait_sc(nxt)
                pend_next = fire_gather(q + 1, nxt)
            else:
                pend_next = None
            for cp in pend:
                cp.wait()
            hbp = hbufs[cur]
            mbp = mbufs[cur % 2]

            @plsc.parallel_loop(0, epc, unroll=2)
            def _edge(k):
                ke = q * epc + k
                al16 = exb[ke, :] * invb[ke, :]
                plsc.store_scatter(alb, [ke * 8 + lane], al16,
                                   mask=lane_mask)
                if hc == 64:
                    for j in range(nh):
                        sc1 = lax.broadcast(al16[2 * j], (16,))
                        sc2 = lax.broadcast(al16[2 * j + 1], (16,))
                        sc = jnp.where(lane_mask, sc1, sc2)
                        hbp[k, pl.ds(j * 16, 16)] = \
                            hbp[k, pl.ds(j * 16, 16)] * sc
                else:
                    m16 = lax.broadcast(al16[0], (16,)) * hbp[k, pl.ds(0, 16)]
                    for j in range(1, nh):
                        m16 = m16 + lax.broadcast(al16[j], (16,)) * \
                            hbp[k, pl.ds(j * 16, 16)]
                    mbp[k, :] = m16

            srcb = hbp if hc == 64 else mbp
            cps = []
            for j in range(sgrp):
                cps.append(pltpu.async_copy(
                    srcb.at[pl.ds(j * SUB, SUB), :],
                    acc_sp.at[dv[c & 1].at[q * sgrp + j]], ssem, add=True))
            prev_sc[cur] = cps
            pend = pend_next
            hb = nxt

        @pl.when(base + B <= E)
        def _full():
            pltpu.sync_copy(alb, alpha_hbm.at[pl.ds(base * 8, B * 8)])

        @pl.when(jnp.logical_and(base < E, base + B > E))
        def _part():
            pltpu.sync_copy(alb.at[pl.ds(0, 512 * 8)],
                            alpha_hbm.at[pl.ds(base * 8, 512 * 8)])

    for i in range(nhb):
        wait_sc(i)
    plsc.subcore_barrier()
    pltpu.sync_copy(acc_sp.at[pl.ds(sid * N_TILE, N_TILE), :],
                    acc_hbm.at[cid, pl.ds(sid * N_TILE, N_TILE), :])


def _pass2(hc, h, inv, ex, src3d, dst3d):
    B = 1024
    epc = 128 if hc == 64 else 256
    acw = 64 if hc == 64 else 16
    mshape = (epc, 16)
    h2shape = (epc, hc) if hc == 64 else (1, hc)
    kern = pl.kernel(
        functools.partial(_pass2_body, hc),
        out_type=[jax.ShapeDtypeStruct((E * 8,), jnp.float32),
                  jax.ShapeDtypeStruct((NC, NP, acw), jnp.float32)],
        mesh=_mesh,
        compiler_params=_params,
        scratch_types=[
            pltpu.VMEM((B // SUB, SUB), jnp.int32),
            pltpu.VMEM((B // SUB, SUB), jnp.int32),
            pltpu.VMEM((B // SUB, SUB), jnp.int32),
            pltpu.VMEM((epc, hc), jnp.float32),
            pltpu.VMEM((epc, hc), jnp.float32),
            pltpu.VMEM(h2shape, jnp.float32),
            pltpu.VMEM(mshape, jnp.float32),
            pltpu.VMEM(mshape, jnp.float32),
            pltpu.VMEM((B, 16), jnp.float32),
            pltpu.VMEM((B, 16), jnp.float32),
            pltpu.VMEM((B * 8,), jnp.float32),
            pltpu.VMEM_SHARED((NP, acw), jnp.float32),
            pltpu.SemaphoreType.DMA,
            pltpu.SemaphoreType.DMA,
            pltpu.SemaphoreType.DMA,
        ],
    )
    return kern(h, inv, ex, src3d, dst3d)


def _k0_body(x_ref, w_ref, asa_ref, ada_ref, h_ref, sa_ref, da_ref):
    h = jnp.dot(x_ref[...], w_ref[...], preferred_element_type=jnp.float32)
    h_ref[...] = h
    sa_ref[...] = jnp.dot(h, asa_ref[...], preferred_element_type=jnp.float32)
    da_ref[...] = jnp.dot(h, ada_ref[...], preferred_element_type=jnp.float32)


def _k0(x, w, asa, ada):
    blk = 2000
    grid = (N + blk - 1) // blk
    d_in, d_out = w.shape
    return pl.pallas_call(
        _k0_body,
        out_shape=[jax.ShapeDtypeStruct((N, d_out), jnp.float32),
                   jax.ShapeDtypeStruct((N, 16), jnp.float32),
                   jax.ShapeDtypeStruct((N, 16), jnp.float32)],
        grid=(grid,),
        in_specs=[pl.BlockSpec((blk, d_in), lambda i: (i, 0)),
                  pl.BlockSpec((d_in, d_out), lambda i: (0, 0)),
                  pl.BlockSpec((d_out, 16), lambda i: (0, 0)),
                  pl.BlockSpec((d_out, 16), lambda i: (0, 0))],
        out_specs=[pl.BlockSpec((blk, d_out), lambda i: (i, 0)),
                   pl.BlockSpec((blk, 16), lambda i: (i, 0)),
                   pl.BlockSpec((blk, 16), lambda i: (i, 0))],
    )(x, w, asa, ada)


def _inv_body(den_ref, inv_ref):
    inv_ref[...] = 1.0 / (den_ref[0] + den_ref[1] + 1e-16)


def _invden(den):
    blk = 2048
    grid = NP // blk
    return pl.pallas_call(
        _inv_body,
        out_shape=jax.ShapeDtypeStruct((NP, 16), jnp.float32),
        grid=(grid,),
        in_specs=[pl.BlockSpec((NC, blk, 16), lambda i: (0, i, 0))],
        out_specs=pl.BlockSpec((blk, 16), lambda i: (i, 0)),
    )(den)


def _k4_body(acc_ref, b1_ref, w2_ref, asa_ref, ada_ref,
             h2_ref, sa_ref, da_ref):
    s = acc_ref[0] + acc_ref[1] + b1_ref[...]
    h1 = jnp.where(s > 0, s, jnp.exp(jnp.minimum(s, 0.0)) - 1.0)
    h2 = jnp.dot(h1, w2_ref[...], preferred_element_type=jnp.float32)
    h2_ref[...] = h2
    sa_ref[...] = jnp.dot(h2, asa_ref[...], preferred_element_type=jnp.float32)
    da_ref[...] = jnp.dot(h2, ada_ref[...], preferred_element_type=jnp.float32)


def _k4(acc, b1, w2, asa, ada):
    blk = 2048
    grid = NP // blk
    return pl.pallas_call(
        _k4_body,
        out_shape=[jax.ShapeDtypeStruct((NP, 128), jnp.float32),
                   jax.ShapeDtypeStruct((NP, 16), jnp.float32),
                   jax.ShapeDtypeStruct((NP, 16), jnp.float32)],
        grid=(grid,),
        in_specs=[pl.BlockSpec((NC, blk, 64), lambda i: (0, i, 0)),
                  pl.BlockSpec((1, 64), lambda i: (0, 0)),
                  pl.BlockSpec((64, 128), lambda i: (0, 0)),
                  pl.BlockSpec((128, 16), lambda i: (0, 0)),
                  pl.BlockSpec((128, 16), lambda i: (0, 0))],
        out_specs=[pl.BlockSpec((blk, 128), lambda i: (i, 0)),
                   pl.BlockSpec((blk, 16), lambda i: (i, 0)),
                   pl.BlockSpec((blk, 16), lambda i: (i, 0))],
    )(acc, b1, w2, asa, ada)


def _k5_body(acc_ref, b2_ref, out_ref):
    s = (acc_ref[0] + acc_ref[1]) * (1.0 / H) + b2_ref[...]
    m = jnp.max(s, axis=-1, keepdims=True)
    lse = jnp.log(jnp.sum(jnp.exp(s - m), axis=-1, keepdims=True)) + m
    out_ref[...] = s - lse


def _k5(acc, b2):
    blk = 2000
    grid = (N + blk - 1) // blk
    return pl.pallas_call(
        _k5_body,
        out_shape=jax.ShapeDtypeStruct((N, 16), jnp.float32),
        grid=(grid,),
        in_specs=[pl.BlockSpec((NC, blk, 16), lambda i: (0, i, 0)),
                  pl.BlockSpec((1, 16), lambda i: (0, 0))],
        out_specs=pl.BlockSpec((blk, 16), lambda i: (i, 0)),
    )(acc, b2)


def _att_mats(att_src, att_dst, ch):
    hc = H * ch
    rows = jnp.arange(hc, dtype=jnp.int32)
    hd = rows // ch
    asa = jnp.zeros((hc, 16), jnp.float32)
    asa = asa.at[rows, hd].set(att_src.reshape(hc))
    asa = asa.at[rows, hd + 8].set(att_dst.reshape(hc))
    ada = jnp.zeros((hc, 16), jnp.float32)
    ada = ada.at[rows, hd].set(att_dst.reshape(hc))
    ada = ada.at[rows, hd + 8].set(att_src.reshape(hc))
    return asa, ada


def kernel(x, edge_index, W1, att_src1, att_dst1, b1,
           W2, att_src2, att_dst2, b2):
    src = edge_index[0]
    dst = edge_index[1]
    padlen = EP - E
    padidx = (jnp.arange(padlen, dtype=jnp.int32) * 37) % N
    src3d = jnp.concatenate([src, padidx]).reshape(EP // IG, IGS, SUB)
    dst3d = jnp.concatenate([dst, padidx]).reshape(EP // IG, IGS, SUB)

    asa1, ada1 = _att_mats(att_src1, att_dst1, 8)
    asa2, ada2 = _att_mats(att_src2, att_dst2, 16)

    h1, sa1, da1 = _k0(x, W1, asa1, ada1)
    ex1, den1 = _pass1(sa1, da1, src3d, dst3d)
    inv1 = _invden(den1)
    alpha1f, acc1 = _pass2(64, h1, inv1, ex1, src3d, dst3d)
    h2, sa2, da2 = _k4(acc1, b1.reshape(1, 64), W2, asa2, ada2)
    ex2, den2 = _pass1(sa2, da2, src3d, dst3d)
    inv2 = _invden(den2)
    alpha2f, acc2 = _pass2(128, h2, inv2, ex2, src3d, dst3d)
    out = _k5(acc2, b2.reshape(1, 16))

    alpha1 = alpha1f.reshape(E, 8)
    alpha2 = alpha2f.reshape(E, 8)
    return (out, (edge_index, alpha1), (edge_index, alpha2))


# speedup vs baseline: 119.9140x; 1.1109x over previous
import functools

import numpy as np
import jax
import jax.numpy as jnp
from jax import lax
from jax.experimental import pallas as pl
from jax.experimental.pallas import tpu as pltpu
from jax.experimental.pallas import tpu_sc as plsc

N = 10000
NP = 10240
E = 320000
D_IN = 128
H = 8
NC = 2
NS = 16
NW = NC * NS
SUB = 128
W_PER = 10240
EP = W_PER * NW
N_TILE = NP // NS
B = 1024
GRP = B // SUB
NCHUNK = W_PER // B

_mesh = plsc.VectorSubcoreMesh(core_axis_name="c", subcore_axis_name="s",
                               num_cores=NC, num_subcores=NS)
_params = pltpu.CompilerParams(use_tc_tiling_on_sc=False,
                               needs_layout_passes=False)

_GDN = lax.GatherDimensionNumbers(offset_dims=(), collapsed_slice_dims=(0,),
                                  start_index_map=(0,))


def _perm16(v, idx_vec):
    return lax.gather(v, idx_vec[:, None], _GDN, (1,),
                      mode=lax.GatherScatterMode.PROMISE_IN_BOUNDS)


def _wid():
    cid = lax.axis_index("c")
    sid = lax.axis_index("s")
    return cid, sid, sid * NC + cid


def _pass1_body(sa_hbm, da_hbm, src3d_hbm, dst3d_hbm, ex_hbm, den_hbm,
                srcv, dv0, dv1, rs0, rs1, rd0, rd1, ex0, ex1,
                den_sp, gsem, ssem):
    cid, sid, wid = _wid()
    base_w = wid * W_PER
    dv = [dv0, dv1]
    rs = [rs0, rs1]
    rd = [rd0, rd1]
    exs = [ex0, ex1]
    lane = lax.iota(jnp.int32, 16)
    rowoff = lane >> 3
    cpat = lane & 7

    zero16 = jnp.zeros((16,), jnp.float32)

    @plsc.parallel_loop(0, N_TILE // 2, unroll=4)
    def _z(r):
        plsc.store_scatter(ex0, [2 * r + rowoff, cpat], zero16)
    pltpu.sync_copy(ex0.at[pl.ds(0, N_TILE), :],
                    den_sp.at[pl.ds(sid * N_TILE, N_TILE), :])
    plsc.subcore_barrier()

    def stage_idx(c, p):
        g = (base_w + c * B) // B
        pltpu.sync_copy(src3d_hbm.at[g], srcv)
        pltpu.sync_copy(dst3d_hbm.at[g], dv[p])

    def fire_gathers(p):
        cps = []
        for j in range(GRP):
            cps.append(pltpu.async_copy(
                sa_hbm.at[srcv.at[j]],
                rs[p].at[pl.ds(j * SUB, SUB), :], gsem))
            cps.append(pltpu.async_copy(
                da_hbm.at[dv[p].at[j]],
                rd[p].at[pl.ds(j * SUB, SUB), :], gsem))
        return cps

    stage_idx(0, 0)
    pend_g = fire_gathers(0)
    prev_sc = [None, None]
    for c in range(NCHUNK):
        p = c & 1
        base = base_w + c * B
        for cp in pend_g:
            cp.wait()
        if c + 1 < NCHUNK:
            if prev_sc[1 - p] is not None:
                for cp in prev_sc[1 - p]:
                    cp.wait()
                prev_sc[1 - p] = None
            stage_idx(c + 1, 1 - p)
            pend_g = fire_gathers(1 - p)
        if prev_sc[p] is not None:
            for cp in prev_sc[p]:
                cp.wait()
            prev_sc[p] = None
        rsp, rdp, exp_ = rs[p], rd[p], exs[p]

        @plsc.parallel_loop(0, B // 2, unroll=2)
        def _pair(k):
            rows = 2 * k + rowoff
            es = plsc.load_gather(rsp, [rows, cpat])
            ed = plsc.load_gather(rdp, [rows, cpat])
            e16 = es + ed
            e16 = jnp.maximum(e16, e16 * 0.2)
            ex16 = jnp.exp(e16)
            valid = (base + 2 * k) < E
            ex16 = jnp.where(valid, ex16, 0.0)
            plsc.store_scatter(exp_, [rows, cpat], ex16)

        pltpu.sync_copy(exp_.at[pl.ds(0, B), :],
                        ex_hbm.at[pl.ds(base, B), :])
        cps = []
        for j in range(GRP):
            cps.append(pltpu.async_copy(
                exp_.at[pl.ds(j * SUB, SUB), :],
                den_sp.at[dv[p].at[j]], ssem, add=True))
        prev_sc[p] = cps

    for q in (0, 1):
        if prev_sc[q] is not None:
            for cp in prev_sc[q]:
                cp.wait()
    plsc.subcore_barrier()
    pltpu.sync_copy(den_sp.at[pl.ds(sid * N_TILE, N_TILE), :],
                    den_hbm.at[cid, pl.ds(sid * N_TILE, N_TILE), :])


def _pass1(sa, da, src3d, dst3d):
    kern = pl.kernel(
        _pass1_body,
        out_type=[jax.ShapeDtypeStruct((EP, 8), jnp.float32),
                  jax.ShapeDtypeStruct((NC, NP, 8), jnp.float32)],
        mesh=_mesh,
        compiler_params=_params,
        scratch_types=[
            pltpu.VMEM((GRP, SUB), jnp.int32),
            pltpu.VMEM((GRP, SUB), jnp.int32),
            pltpu.VMEM((GRP, SUB), jnp.int32),
            pltpu.VMEM((B, 8), jnp.float32),
            pltpu.VMEM((B, 8), jnp.float32),
            pltpu.VMEM((B, 8), jnp.float32),
            pltpu.VMEM((B, 8), jnp.float32),
            pltpu.VMEM((B, 8), jnp.float32),
            pltpu.VMEM((B, 8), jnp.float32),
            pltpu.VMEM_SHARED((NP, 8), jnp.float32),
            pltpu.SemaphoreType.DMA,
            pltpu.SemaphoreType.DMA,
        ],
    )
    return kern(sa, da, src3d, dst3d)


def _pass2_body(hc, h_hbm, inv_hbm, ex_hbm, src3d_hbm, dst3d_hbm,
                alpha_hbm, acc_hbm,
                srcv, dv0, dv1, h0, h1, h2, m0, m1, invb, exb, alb,
                acc_sp, gsem, ssem, isem):
    nh = hc // 16
    epc = 256
    nq = B // epc
    sgrp = epc // SUB
    acw = 64 if hc == 64 else 16
    cid, sid, wid = _wid()
    base_w = wid * W_PER
    lane = lax.iota(jnp.int32, 16)
    rowoff = lane >> 3
    cpat = lane & 7
    dv = [dv0, dv1]
    hbufs = [h0, h1, h2] if hc == 64 else [h0, h1]
    mbufs = [m0, m1]
    nhb = len(hbufs)
    zrows = N_TILE // 5

    @plsc.parallel_loop(0, zrows, unroll=2)
    def _z(r):
        for j in range(acw // 16):
            h0[r, pl.ds(j * 16, 16)] = jnp.zeros((16,), jnp.float32)
    for t in range(5):
        pltpu.sync_copy(
            h0.at[pl.ds(0, zrows), pl.ds(0, acw)],
            acc_sp.at[pl.ds(sid * N_TILE + t * zrows, zrows), :])
    plsc.subcore_barrier()

    prev_sc = [None] * nhb

    def wait_sc(i):
        if prev_sc[i] is not None:
            for cp in prev_sc[i]:
                cp.wait()
            prev_sc[i] = None

    def fire_gather(q, hb):
        cps = []
        for j in range(sgrp):
            cps.append(pltpu.async_copy(
                h_hbm.at[srcv.at[q * sgrp + j]],
                hbufs[hb].at[pl.ds(j * SUB, SUB), :], gsem))
        return cps

    def stage_chunk(c):
        base = base_w + c * B
        p = c & 1
        g = base // B
        pltpu.sync_copy(src3d_hbm.at[g], srcv)
        pltpu.sync_copy(dst3d_hbm.at[g], dv[p])
        pltpu.sync_copy(ex_hbm.at[pl.ds(base, B), :], exb)
        cps = []
        for j in range(GRP):
            cps.append(pltpu.async_copy(
                inv_hbm.at[dv[p].at[j]],
                invb.at[pl.ds(j * SUB, SUB), :], isem))
        for cp in cps:
            cp.wait()

    hb = 0
    for c in range(NCHUNK):
        base = base_w + c * B
        stage_chunk(c)
        pend = None
        for q in range(nq):
            cur = hb
            if pend is None:
                wait_sc(cur)
                pend = fire_gather(q, cur)
            nxt = (cur + 1) % nhb
            if q + 1 < nq:
                wait_sc(nxt)
                pend_next = fire_gather(q + 1, nxt)
            else:
                pend_next = None
            for cp in pend:
                cp.wait()
            hbp = hbufs[cur]
            mbp = mbufs[cur % 2]

            @plsc.parallel_loop(0, epc // 2, unroll=1)
            def _pair(k):
                ke = q * epc + 2 * k
                rows = ke + rowoff
                ex16 = plsc.load_gather(exb, [rows, cpat])
                iv16 = plsc.load_gather(invb, [rows, cpat])
                al16 = ex16 * iv16
                alb[pl.ds(ke * 8, 16)] = al16
                if hc == 64:
                    for j in range(nh):
                        scA = _perm16(al16, rowoff + 2 * j)
                        scB = _perm16(al16, rowoff + (8 + 2 * j))
                        hbp[2 * k, pl.ds(j * 16, 16)] = \
                            hbp[2 * k, pl.ds(j * 16, 16)] * scA
                        hbp[2 * k + 1, pl.ds(j * 16, 16)] = \
                            hbp[2 * k + 1, pl.ds(j * 16, 16)] * scB
                else:
                    mA = lax.broadcast(al16[0], (16,)) * hbp[2 * k, pl.ds(0, 16)]
                    mB = lax.broadcast(al16[8], (16,)) * \
                        hbp[2 * k + 1, pl.ds(0, 16)]
                    for j in range(1, nh):
                        mA = mA + lax.broadcast(al16[j], (16,)) * \
                            hbp[2 * k, pl.ds(j * 16, 16)]
                        mB = mB + lax.broadcast(al16[8 + j], (16,)) * \
                            hbp[2 * k + 1, pl.ds(j * 16, 16)]
                    mbp[2 * k, :] = mA
                    mbp[2 * k + 1, :] = mB

            srcb = hbp if hc == 64 else mbp
            cps = []
            for j in range(sgrp):
                cps.append(pltpu.async_copy(
                    srcb.at[pl.ds(j * SUB, SUB), :],
                    acc_sp.at[dv[c & 1].at[q * sgrp + j]], ssem, add=True))
            prev_sc[cur] = cps
            pend = pend_next
            hb = nxt

        @pl.when(base + B <= E)
        def _full():
            pltpu.sync_copy(alb, alpha_hbm.at[pl.ds(base * 8, B * 8)])

        @pl.when(jnp.logical_and(base < E, base + B > E))
        def _part():
            pltpu.sync_copy(alb.at[pl.ds(0, 512 * 8)],
                            alpha_hbm.at[pl.ds(base * 8, 512 * 8)])

    for i in range(nhb):
        wait_sc(i)
    plsc.subcore_barrier()
    pltpu.sync_copy(acc_sp.at[pl.ds(sid * N_TILE, N_TILE), :],
                    acc_hbm.at[cid, pl.ds(sid * N_TILE, N_TILE), :])


def _pass2(hc, h, inv, ex, src3d, dst3d):
    epc = 256
    acw = 64 if hc == 64 else 16
    mshape = (epc, 16) if hc == 128 else (1, 16)
    h2shape = (epc, hc) if hc == 64 else (1, hc)
    kern = pl.kernel(
        functools.partial(_pass2_body, hc),
        out_type=[jax.ShapeDtypeStruct((E * 8,), jnp.float32),
                  jax.ShapeDtypeStruct((NC, NP, acw), jnp.float32)],
        mesh=_mesh,
        compiler_params=_params,
        scratch_types=[
            pltpu.VMEM((GRP, SUB), jnp.int32),
            pltpu.VMEM((GRP, SUB), jnp.int32),
            pltpu.VMEM((GRP, SUB), jnp.int32),
            pltpu.VMEM((epc, hc), jnp.float32),
            pltpu.VMEM((epc, hc), jnp.float32),
            pltpu.VMEM(h2shape, jnp.float32),
            pltpu.VMEM(mshape, jnp.float32),
            pltpu.VMEM(mshape, jnp.float32),
            pltpu.VMEM((B, 8), jnp.float32),
            pltpu.VMEM((B, 8), jnp.float32),
            pltpu.VMEM((B * 8,), jnp.float32),
            pltpu.VMEM_SHARED((NP, acw), jnp.float32),
            pltpu.SemaphoreType.DMA,
            pltpu.SemaphoreType.DMA,
            pltpu.SemaphoreType.DMA,
        ],
    )
    return kern(h, inv, ex, src3d, dst3d)


def _k0_body(x_ref, w_ref, as_ref, ad_ref, h_ref, sa_ref, da_ref):
    h = jnp.dot(x_ref[...], w_ref[...], preferred_element_type=jnp.float32)
    h_ref[...] = h
    sa_ref[...] = jnp.dot(h, as_ref[...], preferred_element_type=jnp.float32)
    da_ref[...] = jnp.dot(h, ad_ref[...], preferred_element_type=jnp.float32)


def _k0(x, w, a_s, a_d):
    blk = 2000
    grid = (N + blk - 1) // blk
    d_in, d_out = w.shape
    return pl.pallas_call(
        _k0_body,
        out_shape=[jax.ShapeDtypeStruct((N, d_out), jnp.float32),
                   jax.ShapeDtypeStruct((N, 8), jnp.float32),
                   jax.ShapeDtypeStruct((N, 8), jnp.float32)],
        grid=(grid,),
        in_specs=[pl.BlockSpec((blk, d_in), lambda i: (i, 0)),
                  pl.BlockSpec((d_in, d_out), lambda i: (0, 0)),
                  pl.BlockSpec((d_out, 8), lambda i: (0, 0)),
                  pl.BlockSpec((d_out, 8), lambda i: (0, 0))],
        out_specs=[pl.BlockSpec((blk, d_out), lambda i: (i, 0)),
                   pl.BlockSpec((blk, 8), lambda i: (i, 0)),
                   pl.BlockSpec((blk, 8), lambda i: (i, 0))],
    )(x, w, a_s, a_d)


def _inv_body(den_ref, inv_ref):
    inv_ref[...] = 1.0 / (den_ref[0] + den_ref[1] + 1e-16)


def _invden(den):
    blk = 2048
    grid = NP // blk
    return pl.pallas_call(
        _inv_body,
        out_shape=jax.ShapeDtypeStruct((NP, 8), jnp.float32),
        grid=(grid,),
        in_specs=[pl.BlockSpec((NC, blk, 8), lambda i: (0, i, 0))],
        out_specs=pl.BlockSpec((blk, 8), lambda i: (i, 0)),
    )(den)


def _k4_body(acc_ref, b1_ref, w2_ref, as_ref, ad_ref,
             h2_ref, sa_ref, da_ref):
    s = acc_ref[0] + acc_ref[1] + b1_ref[...]
    h1 = jnp.where(s > 0, s, jnp.exp(jnp.minimum(s, 0.0)) - 1.0)
    h2 = jnp.dot(h1, w2_ref[...], preferred_element_type=jnp.float32)
    h2_ref[...] = h2
    sa_ref[...] = jnp.dot(h2, as_ref[...], preferred_element_type=jnp.float32)
    da_ref[...] = jnp.dot(h2, ad_ref[...], preferred_element_type=jnp.float32)


def _k4(acc, b1, w2, a_s, a_d):
    blk = 2048
    grid = NP // blk
    return pl.pallas_call(
        _k4_body,
        out_shape=[jax.ShapeDtypeStruct((NP, 128), jnp.float32),
                   jax.ShapeDtypeStruct((NP, 8), jnp.float32),
                   jax.ShapeDtypeStruct((NP, 8), jnp.float32)],
        grid=(grid,),
        in_specs=[pl.BlockSpec((NC, blk, 64), lambda i: (0, i, 0)),
                  pl.BlockSpec((1, 64), lambda i: (0, 0)),
                  pl.BlockSpec((64, 128), lambda i: (0, 0)),
                  pl.BlockSpec((128, 8), lambda i: (0, 0)),
                  pl.BlockSpec((128, 8), lambda i: (0, 0))],
        out_specs=[pl.BlockSpec((blk, 128), lambda i: (i, 0)),
                   pl.BlockSpec((blk, 8), lambda i: (i, 0)),
                   pl.BlockSpec((blk, 8), lambda i: (i, 0))],
    )(acc, b1, w2, a_s, a_d)


def _k5_body(acc_ref, b2_ref, out_ref):
    s = (acc_ref[0] + acc_ref[1]) * (1.0 / H) + b2_ref[...]
    m = jnp.max(s, axis=-1, keepdims=True)
    lse = jnp.log(jnp.sum(jnp.exp(s - m), axis=-1, keepdims=True)) + m
    out_ref[...] = s - lse


def _k5(acc, b2):
    blk = 2000
    grid = (N + blk - 1) // blk
    return pl.pallas_call(
        _k5_body,
        out_shape=jax.ShapeDtypeStruct((N, 16), jnp.float32),
        grid=(grid,),
        in_specs=[pl.BlockSpec((NC, blk, 16), lambda i: (0, i, 0)),
                  pl.BlockSpec((1, 16), lambda i: (0, 0))],
        out_specs=pl.BlockSpec((blk, 16), lambda i: (i, 0)),
    )(acc, b2)


def _att_mats(att_src, att_dst, ch):
    hc = H * ch
    rows = jnp.arange(hc, dtype=jnp.int32)
    hd = rows // ch
    a_s = jnp.zeros((hc, 8), jnp.float32)
    a_s = a_s.at[rows, hd].set(att_src.reshape(hc))
    a_d = jnp.zeros((hc, 8), jnp.float32)
    a_d = a_d.at[rows, hd].set(att_dst.reshape(hc))
    return a_s, a_d


def kernel(x, edge_index, W1, att_src1, att_dst1, b1,
           W2, att_src2, att_dst2, b2):
    src = edge_index[0]
    dst = edge_index[1]
    padlen = EP - E
    padidx = (jnp.arange(padlen, dtype=jnp.int32) * 37) % N
    src3d = jnp.concatenate([src, padidx]).reshape(EP // B, GRP, SUB)
    dst3d = jnp.concatenate([dst, padidx]).reshape(EP // B, GRP, SUB)

    as1, ad1 = _att_mats(att_src1, att_dst1, 8)
    as2, ad2 = _att_mats(att_src2, att_dst2, 16)

    h1, sa1, da1 = _k0(x, W1, as1, ad1)
    ex1, den1 = _pass1(sa1, da1, src3d, dst3d)
    inv1 = _invden(den1)
    alpha1f, acc1 = _pass2(64, h1, inv1, ex1, src3d, dst3d)
    h2, sa2, da2 = _k4(acc1, b1.reshape(1, 64), W2, as2, ad2)
    ex2, den2 = _pass1(sa2, da2, src3d, dst3d)
    inv2 = _invden(den2)
    alpha2f, acc2 = _pass2(128, h2, inv2, ex2, src3d, dst3d)
    out = _k5(acc2, b2.reshape(1, 16))

    alpha1 = alpha1f.reshape(E, 8)
    alpha2 = alpha2f.reshape(E, 8)
    return (out, (edge_index, alpha1), (edge_index, alpha2))
